# baseline (speedup 1.0000x reference)
import functools

import jax
import jax.numpy as jnp
from jax import lax
from jax.experimental import pallas as pl
from jax.experimental.pallas import tpu as pltpu
from jax.experimental.pallas import tpu_sc as plsc


def _make_sc_kernel(b: int, h: int, d: int, r_chunk: int):
    info = plsc.get_sparse_core_info()
    nc, ns = info.num_cores, info.num_subcores
    nw = nc * ns
    assert b % nw == 0
    rows_per_w = b // nw
    assert rows_per_w % r_chunk == 0
    n_chunks = rows_per_w // r_chunk
    assert d == 32 and h % 8 == 0
    n_full = (h // 16) * 16
    mesh = plsc.VectorSubcoreMesh(core_axis_name="c", subcore_axis_name="s")

    @functools.partial(
        pl.kernel,
        mesh=mesh,
        out_type=jax.ShapeDtypeStruct((b, h, d), jnp.float32),
        compiler_params=pltpu.CompilerParams(use_tc_tiling_on_sc=False),
        scratch_types=[
            pltpu.VMEM((r_chunk, h), jnp.int32),
            pltpu.VMEM((r_chunk, h), jnp.float32),
            pltpu.VMEM((r_chunk * h, d), jnp.float32),
            pltpu.SemaphoreType.DMA,
        ],
    )
    def sc_kernel(idx_hbm, score_hbm, table_hbm, out_hbm,
                  idx_v, score_v, rows_v, sem):
        wid = lax.axis_index("s") * nc + lax.axis_index("c")
        row0 = wid * rows_per_w

        def chunk_body(g, carry):
            off = row0 + g * r_chunk
            pltpu.sync_copy(idx_hbm.at[pl.ds(off, r_chunk)], idx_v)
            pltpu.sync_copy(score_hbm.at[pl.ds(off, r_chunk)], score_v)
            descs = [
                pltpu.async_copy(table_hbm.at[idx_v.at[r]],
                                 rows_v.at[pl.ds(r * h, h)], sem)
                for r in range(r_chunk)
            ]
            for dsc in descs:
                dsc.wait()

            def row_body(a, c):
                def grp_body(q, c2):
                    cb = q * 16
                    svec = score_v[a, pl.ds(cb, 16)]
                    fl = a * h + cb
                    for j in range(16):
                        s = svec[j]
                        rows_v[fl + j, pl.ds(0, 16)] = (
                            rows_v[fl + j, pl.ds(0, 16)] * s)
                        rows_v[fl + j, pl.ds(16, 16)] = (
                            rows_v[fl + j, pl.ds(16, 16)] * s)
                    return c2

                lax.fori_loop(0, h // 16, grp_body, 0)
                if n_full != h:
                    svec = score_v[a, pl.ds(h - 16, 16)]
                    fl = a * h + (h - 16)
                    for j in range(16 - (h - n_full), 16):
                        s = svec[j]
                        rows_v[fl + j, pl.ds(0, 16)] = (
                            rows_v[fl + j, pl.ds(0, 16)] * s)
                        rows_v[fl + j, pl.ds(16, 16)] = (
                            rows_v[fl + j, pl.ds(16, 16)] * s)
                return c

            lax.fori_loop(0, r_chunk, row_body, 0)
            for r in range(r_chunk):
                pltpu.sync_copy(rows_v.at[pl.ds(r * h, h)], out_hbm.at[off + r])
            return carry

        lax.fori_loop(0, n_chunks, chunk_body, 0)

    return sc_kernel


def kernel(v_e, v_score, table):
    b, h = v_e.shape
    v, d = table.shape
    idx = v_e.astype(jnp.int32)
    score = v_score.astype(jnp.float32)
    return _make_sc_kernel(b, h, d, r_chunk=16)(idx, score, table)
